# baseline (speedup 1.0000x reference)
import functools

import jax
import jax.numpy as jnp
from jax import lax
from jax.experimental import pallas as pl
from jax.experimental.pallas import tpu as pltpu
from jax.experimental.pallas import tpu_sc as plsc

D_MODEL = 768
N_PATCHES = 576
N_TOT = N_PATCHES + 1
BATCH = 64

_NUM_CORES = 2
_NUM_SUBCORES = 16
_NUM_WORKERS = _NUM_CORES * _NUM_SUBCORES
_TCHUNK = N_PATCHES // _NUM_WORKERS
_LANES = 16
_VPR = D_MODEL // _LANES
_CHUNK = _TCHUNK * D_MODEL


def _sc_body(in_hbm, cls_hbm, pos_hbm, out_hbm, pos_v, buf_v, cls_v, sem):
    c = lax.axis_index("c")
    s = lax.axis_index("s")
    wid = c * _NUM_SUBCORES + s
    t0 = wid * _TCHUNK

    pltpu.sync_copy(pos_hbm.at[pl.ds((t0 + 1) * D_MODEL, _CHUNK)], pos_v)

    @pl.when(wid == 0)
    def _():
        pltpu.sync_copy(cls_hbm, cls_v)
        pltpu.sync_copy(pos_hbm.at[pl.ds(0, D_MODEL)],
                        buf_v.at[pl.ds(0, D_MODEL)])
        for v in range(_VPR):
            sl = pl.ds(v * _LANES, _LANES)
            cls_v[sl] = cls_v[sl] + buf_v[sl]
        handles = [
            pltpu.async_copy(
                cls_v, out_hbm.at[pl.ds(b * N_TOT * D_MODEL, D_MODEL)], sem)
            for b in range(BATCH)
        ]
        for h in handles:
            h.wait()

    def b_body(b, carry):
        base_in = (b * N_PATCHES + t0) * D_MODEL
        base_out = (b * N_TOT + t0 + 1) * D_MODEL
        pltpu.sync_copy(in_hbm.at[pl.ds(base_in, _CHUNK)], buf_v)

        def r_body(r, carry2):
            base = r * D_MODEL
            for v in range(_VPR):
                sl = pl.ds(base + v * _LANES, _LANES)
                buf_v[sl] = buf_v[sl] + pos_v[sl]
            return carry2

        lax.fori_loop(0, _TCHUNK, r_body, 0)
        pltpu.sync_copy(buf_v, out_hbm.at[pl.ds(base_out, _CHUNK)])
        return carry

    lax.fori_loop(0, BATCH, b_body, 0)


_sc_call = functools.partial(
    pl.kernel,
    mesh=plsc.VectorSubcoreMesh(core_axis_name="c", subcore_axis_name="s"),
    out_type=jax.ShapeDtypeStruct((BATCH * N_TOT * D_MODEL,), jnp.float32),
    scratch_types=[
        pltpu.VMEM((_CHUNK,), jnp.float32),
        pltpu.VMEM((_CHUNK,), jnp.float32),
        pltpu.VMEM((D_MODEL,), jnp.float32),
        pltpu.SemaphoreType.DMA,
    ],
)(_sc_body)


def kernel(inputs, class_embed, pos_table):
    flat_in = inputs.reshape(BATCH * N_PATCHES * D_MODEL)
    cls = class_embed.reshape(D_MODEL)
    pos = pos_table.reshape(N_TOT * D_MODEL)
    out = _sc_call(flat_in, cls, pos)
    return out.reshape(BATCH, N_TOT, D_MODEL)


# speedup vs baseline: 1.2207x; 1.2207x over previous
import functools

import jax
import jax.numpy as jnp
from jax import lax
from jax.experimental import pallas as pl
from jax.experimental.pallas import tpu as pltpu
from jax.experimental.pallas import tpu_sc as plsc

D_MODEL = 768
N_PATCHES = 576
N_TOT = N_PATCHES + 1
BATCH = 64

_NUM_CORES = 2
_NUM_SUBCORES = 16
_NUM_WORKERS = _NUM_CORES * _NUM_SUBCORES
_TCHUNK = N_PATCHES // _NUM_WORKERS
_LANES = 16
_VPR = D_MODEL // _LANES
_CHUNK = _TCHUNK * D_MODEL
_CLS_PER_W = BATCH // _NUM_WORKERS


def _sc_body(in_hbm, cls_hbm, pos_hbm, out_hbm,
             pos_v, cls_v, tmp_v, in0, in1, ot0, ot1,
             cls_sem, is0, is1, os0, os1):
    c = lax.axis_index("c")
    s = lax.axis_index("s")
    wid = c * _NUM_SUBCORES + s
    t0 = wid * _TCHUNK

    in_bufs = (in0, in1)
    out_bufs = (ot0, ot1)
    in_sems = (is0, is1)
    out_sems = (os0, os1)

    pltpu.sync_copy(pos_hbm.at[pl.ds((t0 + 1) * D_MODEL, _CHUNK)], pos_v)
    pltpu.sync_copy(cls_hbm, cls_v)
    pltpu.sync_copy(pos_hbm.at[pl.ds(0, D_MODEL)], tmp_v)
    for v in range(_VPR):
        sl = pl.ds(v * _LANES, _LANES)
        cls_v[sl] = cls_v[sl] + tmp_v[sl]
    cls_handles = [
        pltpu.async_copy(
            cls_v,
            out_hbm.at[pl.ds((wid * _CLS_PER_W + j) * N_TOT * D_MODEL,
                             D_MODEL)],
            cls_sem)
        for j in range(_CLS_PER_W)
    ]

    def start_in(b, i):
        base_in = (b * N_PATCHES + t0) * D_MODEL
        pltpu.async_copy(in_hbm.at[pl.ds(base_in, _CHUNK)], in_bufs[i],
                         in_sems[i])

    def wait_in(i):
        pltpu.make_async_copy(in_hbm.at[pl.ds(0, _CHUNK)], in_bufs[i],
                              in_sems[i]).wait()

    def start_out(b, i):
        base_out = (b * N_TOT + t0 + 1) * D_MODEL
        pltpu.async_copy(out_bufs[i], out_hbm.at[pl.ds(base_out, _CHUNK)],
                         out_sems[i])

    def wait_out(i):
        pltpu.make_async_copy(out_bufs[i], out_hbm.at[pl.ds(0, _CHUNK)],
                              out_sems[i]).wait()

    start_in(0, 0)
    start_in(1, 1)

    def g_body(g, carry):
        for i in range(2):
            b = g * 2 + i
            wait_in(i)

            @pl.when(g > 0)
            def _():
                wait_out(i)

            def r_body(r, cr):
                base = r * D_MODEL
                for v in range(_VPR):
                    sl = pl.ds(base + v * _LANES, _LANES)
                    out_bufs[i][sl] = in_bufs[i][sl] + pos_v[sl]
                return cr

            lax.fori_loop(0, _TCHUNK, r_body, 0)
            start_out(b, i)

            @pl.when(b + 2 < BATCH)
            def _():
                start_in(b + 2, i)
        return carry

    lax.fori_loop(0, BATCH // 2, g_body, 0)

    wait_out(0)
    wait_out(1)
    for h in cls_handles:
        h.wait()


_sc_call = functools.partial(
    pl.kernel,
    mesh=plsc.VectorSubcoreMesh(core_axis_name="c", subcore_axis_name="s"),
    out_type=jax.ShapeDtypeStruct((BATCH * N_TOT * D_MODEL,), jnp.float32),
    scratch_types=[
        pltpu.VMEM((_CHUNK,), jnp.float32),
        pltpu.VMEM((D_MODEL,), jnp.float32),
        pltpu.VMEM((D_MODEL,), jnp.float32),
        pltpu.VMEM((_CHUNK,), jnp.float32),
        pltpu.VMEM((_CHUNK,), jnp.float32),
        pltpu.VMEM((_CHUNK,), jnp.float32),
        pltpu.VMEM((_CHUNK,), jnp.float32),
        pltpu.SemaphoreType.DMA,
        pltpu.SemaphoreType.DMA,
        pltpu.SemaphoreType.DMA,
        pltpu.SemaphoreType.DMA,
        pltpu.SemaphoreType.DMA,
    ],
)(_sc_body)


def kernel(inputs, class_embed, pos_table):
    flat_in = inputs.reshape(BATCH * N_PATCHES * D_MODEL)
    cls = class_embed.reshape(D_MODEL)
    pos = pos_table.reshape(N_TOT * D_MODEL)
    out = _sc_call(flat_in, cls, pos)
    return out.reshape(BATCH, N_TOT, D_MODEL)


# speedup vs baseline: 1.4862x; 1.2176x over previous
import functools

import jax
import jax.numpy as jnp
from jax import lax
from jax.experimental import pallas as pl
from jax.experimental.pallas import tpu as pltpu
from jax.experimental.pallas import tpu_sc as plsc

D_MODEL = 768
N_PATCHES = 576
N_TOT = N_PATCHES + 1
BATCH = 64

_NUM_CORES = 2
_NUM_SUBCORES = 16
_NUM_WORKERS = _NUM_CORES * _NUM_SUBCORES
_LANES = 16
_VPR = D_MODEL // _LANES

_RCHUNK = 24
_RIN = _RCHUNK + 8
_NCH = N_PATCHES // _RCHUNK
_ITEMS = _NCH * BATCH
_ITEMS_PER_W = _ITEMS // _NUM_WORKERS


def _sc_body(in_hbm, cls_hbm, pos_hbm, out_hbm,
             pos_v, cls_v, in0, in1, ot0, ot1,
             is0, is1, os0, os1):
    c_ax = lax.axis_index("c")
    s_ax = lax.axis_index("s")
    wid = c_ax * _NUM_SUBCORES + s_ax
    item0 = wid * _ITEMS_PER_W

    in_bufs = (in0, in1)
    out_bufs = (ot0, ot1)
    in_sems = (is0, is1)
    out_sems = (os0, os1)

    pltpu.sync_copy(cls_hbm, cls_v)

    def load_pos(c):
        start = pl.multiple_of(c * _RCHUNK, 8)
        pltpu.sync_copy(pos_hbm.at[pl.ds(start, _RCHUNK)], pos_v)

    load_pos(item0 // BATCH)

    def start_in(item, i):
        c = item // BATCH
        b = item % BATCH
        base = pl.multiple_of(b * N_PATCHES + lax.max(c * _RCHUNK - 8, 0), 8)
        pltpu.async_copy(in_hbm.at[pl.ds(base, _RIN)], in_bufs[i], in_sems[i])

    def wait_in(i):
        pltpu.make_async_copy(in_hbm.at[pl.ds(0, _RIN)], in_bufs[i],
                              in_sems[i]).wait()

    def start_out(item, i):
        c = item // BATCH
        b = item % BATCH
        start = pl.multiple_of(c * _RCHUNK, 8)
        pltpu.async_copy(out_bufs[i], out_hbm.at[b, pl.ds(start, _RCHUNK)],
                         out_sems[i])

    def wait_out(i):
        pltpu.make_async_copy(out_bufs[i], out_hbm.at[0, pl.ds(0, _RCHUNK)],
                              out_sems[i]).wait()

    start_in(item0, 0)
    start_in(item0 + 1, 1)

    def g_body(g, prev_c):
        for i in range(2):
            item = item0 + g * 2 + i
            c = item // BATCH
            b = item % BATCH

            @pl.when(c != prev_c)
            def _():
                load_pos(c)

            wait_in(i)

            @pl.when(g > 0)
            def _():
                wait_out(i)

            roff = lax.select(c > 0, 7, -1)
            first = lax.select(c > 0, 7, 0)
            not_cls = c > 0

            for v in range(_VPR):
                sl = pl.ds(v * _LANES, _LANES)
                src = jnp.where(not_cls, in_bufs[i][first, sl], cls_v[0, sl])
                out_bufs[i][0, sl] = src + pos_v[0, sl]

            def r_body(j, cr):
                for v in range(_VPR):
                    sl = pl.ds(v * _LANES, _LANES)
                    out_bufs[i][j, sl] = in_bufs[i][j + roff, sl] + pos_v[j, sl]
                return cr

            lax.fori_loop(1, _RCHUNK, r_body, 0)
            start_out(item, i)

            @pl.when(g * 2 + i + 2 < _ITEMS_PER_W)
            def _():
                start_in(item + 2, i)

            prev_c = c
        return prev_c

    lax.fori_loop(0, _ITEMS_PER_W // 2, g_body, item0 // BATCH)

    wait_out(0)
    wait_out(1)

    pltpu.sync_copy(pos_hbm.at[pl.ds(N_PATCHES, 1)], pos_v.at[pl.ds(0, 1)])
    for j in range(2):
        b = wid * 2 + j
        pltpu.sync_copy(in_hbm.at[pl.ds(b * N_PATCHES + N_PATCHES - 8, 8)],
                        in_bufs[j].at[pl.ds(0, 8)])
        for v in range(_VPR):
            sl = pl.ds(v * _LANES, _LANES)
            out_bufs[j][0, sl] = in_bufs[j][7, sl] + pos_v[0, sl]
        pltpu.sync_copy(out_bufs[j].at[pl.ds(0, 1)],
                        out_hbm.at[b, pl.ds(N_PATCHES, 1)])


_sc_call = functools.partial(
    pl.kernel,
    mesh=plsc.VectorSubcoreMesh(core_axis_name="c", subcore_axis_name="s"),
    out_type=jax.ShapeDtypeStruct((BATCH, N_TOT, D_MODEL), jnp.float32),
    scratch_types=[
        pltpu.VMEM((_RCHUNK, D_MODEL), jnp.float32),
        pltpu.VMEM((1, D_MODEL), jnp.float32),
        pltpu.VMEM((_RIN, D_MODEL), jnp.float32),
        pltpu.VMEM((_RIN, D_MODEL), jnp.float32),
        pltpu.VMEM((_RCHUNK, D_MODEL), jnp.float32),
        pltpu.VMEM((_RCHUNK, D_MODEL), jnp.float32),
        pltpu.SemaphoreType.DMA,
        pltpu.SemaphoreType.DMA,
        pltpu.SemaphoreType.DMA,
        pltpu.SemaphoreType.DMA,
    ],
)(_sc_body)


def kernel(inputs, class_embed, pos_table):
    flat_in = inputs.reshape(BATCH * N_PATCHES, D_MODEL)
    cls = class_embed.reshape(1, D_MODEL)
    out = _sc_call(flat_in, cls, pos_table)
    return out


# speedup vs baseline: 2.6931x; 1.8120x over previous
import functools

import jax
import jax.numpy as jnp
from jax import lax
from jax.experimental import pallas as pl
from jax.experimental.pallas import tpu as pltpu
from jax.experimental.pallas import tpu_sc as plsc

D_MODEL = 768
N_PATCHES = 576
N_TOT = N_PATCHES + 1
BATCH = 64

_NUM_CORES = 2
_NUM_SUBCORES = 16
_NUM_WORKERS = _NUM_CORES * _NUM_SUBCORES
_LANES = 16
_VPR = D_MODEL // _LANES

_RCHUNK = 24
_RIN = _RCHUNK + 8
_NCH = N_PATCHES // _RCHUNK
_ITEMS = _NCH * BATCH
_ITEMS_PER_W = _ITEMS // _NUM_WORKERS


def _sc_body(in_hbm, cls_hbm, pos_hbm, out_hbm,
             pos_v, cls_v, in0, in1, ot0, ot1,
             is0, is1, os0, os1):
    c_ax = lax.axis_index("c")
    s_ax = lax.axis_index("s")
    wid = c_ax * _NUM_SUBCORES + s_ax
    item0 = wid * _ITEMS_PER_W

    in_bufs = (in0, in1)
    out_bufs = (ot0, ot1)
    in_sems = (is0, is1)
    out_sems = (os0, os1)

    pltpu.sync_copy(cls_hbm, cls_v)

    def load_pos(c):
        start = pl.multiple_of(c * _RCHUNK, 8)
        pltpu.sync_copy(pos_hbm.at[pl.ds(start, _RCHUNK)], pos_v)

    load_pos(item0 // BATCH)

    def start_in(item, i):
        c = item // BATCH
        b = item % BATCH
        base = pl.multiple_of(b * N_PATCHES + lax.max(c * _RCHUNK - 8, 0), 8)
        pltpu.async_copy(in_hbm.at[pl.ds(base, _RIN)], in_bufs[i], in_sems[i])

    def wait_in(i):
        pltpu.make_async_copy(in_hbm.at[pl.ds(0, _RIN)], in_bufs[i],
                              in_sems[i]).wait()

    def start_out(item, i):
        c = item // BATCH
        b = item % BATCH
        start = pl.multiple_of(c * _RCHUNK, 8)
        pltpu.async_copy(out_bufs[i], out_hbm.at[b, pl.ds(start, _RCHUNK)],
                         out_sems[i])

    def wait_out(i):
        pltpu.make_async_copy(out_bufs[i], out_hbm.at[0, pl.ds(0, _RCHUNK)],
                              out_sems[i]).wait()

    start_in(item0, 0)
    start_in(item0 + 1, 1)

    def g_body(g, prev_c):
        for i in range(2):
            item = item0 + g * 2 + i
            c = item // BATCH
            b = item % BATCH

            @pl.when(c != prev_c)
            def _():
                load_pos(c)

            wait_in(i)

            @pl.when(g > 0)
            def _():
                wait_out(i)

            roff = lax.select(c > 0, 7, -1)

            def r_body(j, cr):
                jr = lax.max(j + roff, 0)

                @plsc.parallel_loop(0, _VPR, unroll=8)
                def _(v):
                    sl = pl.ds(v * _LANES, _LANES)
                    out_bufs[i][j, sl] = in_bufs[i][jr, sl] + pos_v[j, sl]

                return cr

            lax.fori_loop(0, _RCHUNK, r_body, 0)

            @pl.when(c == 0)
            def _():
                for v in range(_VPR):
                    sl = pl.ds(v * _LANES, _LANES)
                    out_bufs[i][0, sl] = cls_v[0, sl] + pos_v[0, sl]

            start_out(item, i)

            @pl.when(g * 2 + i + 2 < _ITEMS_PER_W)
            def _():
                start_in(item + 2, i)

            prev_c = c
        return prev_c

    lax.fori_loop(0, _ITEMS_PER_W // 2, g_body, item0 // BATCH)

    wait_out(0)
    wait_out(1)

    pltpu.sync_copy(pos_hbm.at[pl.ds(N_PATCHES, 1)], pos_v.at[pl.ds(0, 1)])
    for j in range(2):
        b = wid * 2 + j
        pltpu.sync_copy(in_hbm.at[pl.ds(b * N_PATCHES + N_PATCHES - 8, 8)],
                        in_bufs[j].at[pl.ds(0, 8)])
        for v in range(_VPR):
            sl = pl.ds(v * _LANES, _LANES)
            out_bufs[j][0, sl] = in_bufs[j][7, sl] + pos_v[0, sl]
        pltpu.sync_copy(out_bufs[j].at[pl.ds(0, 1)],
                        out_hbm.at[b, pl.ds(N_PATCHES, 1)])


_sc_call = functools.partial(
    pl.kernel,
    mesh=plsc.VectorSubcoreMesh(core_axis_name="c", subcore_axis_name="s"),
    out_type=jax.ShapeDtypeStruct((BATCH, N_TOT, D_MODEL), jnp.float32),
    scratch_types=[
        pltpu.VMEM((_RCHUNK, D_MODEL), jnp.float32),
        pltpu.VMEM((1, D_MODEL), jnp.float32),
        pltpu.VMEM((_RIN, D_MODEL), jnp.float32),
        pltpu.VMEM((_RIN, D_MODEL), jnp.float32),
        pltpu.VMEM((_RCHUNK, D_MODEL), jnp.float32),
        pltpu.VMEM((_RCHUNK, D_MODEL), jnp.float32),
        pltpu.SemaphoreType.DMA,
        pltpu.SemaphoreType.DMA,
        pltpu.SemaphoreType.DMA,
        pltpu.SemaphoreType.DMA,
    ],
)(_sc_body)


def kernel(inputs, class_embed, pos_table):
    flat_in = inputs.reshape(BATCH * N_PATCHES, D_MODEL)
    cls = class_embed.reshape(1, D_MODEL)
    out = _sc_call(flat_in, cls, pos_table)
    return out


# speedup vs baseline: 2.8669x; 1.0646x over previous
import functools

import jax
import jax.numpy as jnp
from jax import lax
from jax.experimental import pallas as pl
from jax.experimental.pallas import tpu as pltpu
from jax.experimental.pallas import tpu_sc as plsc

D_MODEL = 768
N_PATCHES = 576
N_TOT = N_PATCHES + 1
BATCH = 64

_NUM_CORES = 2
_NUM_SUBCORES = 16
_NUM_WORKERS = _NUM_CORES * _NUM_SUBCORES
_LANES = 16
_VPR = D_MODEL // _LANES

_RCHUNK = 32
_NCH = N_PATCHES // _RCHUNK
_ITEMS = _NCH * BATCH
_ITEMS_PER_W = _ITEMS // _NUM_WORKERS


def _sc_body(in_hbm, cls_hbm, pos_hbm, out_hbm,
             pos_v, cls_v, ix0, ix1, in0, in1, ot0, ot1,
             is0, is1, os0, os1):
    c_ax = lax.axis_index("c")
    s_ax = lax.axis_index("s")
    wid = c_ax * _NUM_SUBCORES + s_ax
    item0 = wid * _ITEMS_PER_W

    in_bufs = (in0, in1)
    out_bufs = (ot0, ot1)
    ix_bufs = (ix0, ix1)
    in_sems = (is0, is1)
    out_sems = (os0, os1)

    iota = lax.iota(jnp.int32, 16)

    pltpu.sync_copy(cls_hbm, cls_v)

    def load_pos(c):
        start = pl.multiple_of(c * _RCHUNK, 8)
        pltpu.sync_copy(pos_hbm.at[pl.ds(start, _RCHUNK)], pos_v)

    load_pos(item0 // BATCH)

    def start_in(item, i):
        c = item // BATCH
        b = item % BATCH
        base = b * N_PATCHES + c * _RCHUNK - 1
        ix_bufs[i][pl.ds(0, _LANES)] = jnp.maximum(base + iota, 0)
        ix_bufs[i][pl.ds(_LANES, _LANES)] = base + _LANES + iota
        pltpu.async_copy(in_hbm.at[ix_bufs[i]], in_bufs[i], in_sems[i])

    def wait_in(i):
        pltpu.make_async_copy(in_hbm.at[ix_bufs[i]], in_bufs[i],
                              in_sems[i]).wait()

    def start_out(item, i):
        c = item // BATCH
        b = item % BATCH
        start = pl.multiple_of(c * _RCHUNK, 8)
        pltpu.async_copy(out_bufs[i], out_hbm.at[b, pl.ds(start, _RCHUNK)],
                         out_sems[i])

    def wait_out(i):
        pltpu.make_async_copy(out_bufs[i], out_hbm.at[0, pl.ds(0, _RCHUNK)],
                              out_sems[i]).wait()

    start_in(item0, 0)
    start_in(item0 + 1, 1)

    def g_body(g, prev_c):
        for i in range(2):
            item = item0 + g * 2 + i
            c = item // BATCH

            @pl.when(c != prev_c)
            def _():
                load_pos(c)

            wait_in(i)

            @pl.when(g > 0)
            def _():
                wait_out(i)

            def r_body(j, cr):
                @plsc.parallel_loop(0, _VPR, unroll=16)
                def _(v):
                    sl = pl.ds(v * _LANES, _LANES)
                    out_bufs[i][j, sl] = in_bufs[i][j, sl] + pos_v[j, sl]

                return cr

            lax.fori_loop(0, _RCHUNK, r_body, 0)

            @pl.when(c == 0)
            def _():
                for v in range(_VPR):
                    sl = pl.ds(v * _LANES, _LANES)
                    out_bufs[i][0, sl] = cls_v[0, sl] + pos_v[0, sl]

            start_out(item, i)

            @pl.when(g * 2 + i + 2 < _ITEMS_PER_W)
            def _():
                start_in(item + 2, i)

            prev_c = c
        return prev_c

    lax.fori_loop(0, _ITEMS_PER_W // 2, g_body, item0 // BATCH)

    wait_out(0)
    wait_out(1)

    pltpu.sync_copy(pos_hbm.at[pl.ds(N_PATCHES, 1)], pos_v.at[pl.ds(0, 1)])
    for j in range(2):
        b = wid * 2 + j
        pltpu.sync_copy(in_hbm.at[pl.ds(b * N_PATCHES + N_PATCHES - 8, 8)],
                        in_bufs[j].at[pl.ds(0, 8)])
        for v in range(_VPR):
            sl = pl.ds(v * _LANES, _LANES)
            out_bufs[j][0, sl] = in_bufs[j][7, sl] + pos_v[0, sl]
        pltpu.sync_copy(out_bufs[j].at[pl.ds(0, 1)],
                        out_hbm.at[b, pl.ds(N_PATCHES, 1)])


_sc_call = functools.partial(
    pl.kernel,
    mesh=plsc.VectorSubcoreMesh(core_axis_name="c", subcore_axis_name="s"),
    out_type=jax.ShapeDtypeStruct((BATCH, N_TOT, D_MODEL), jnp.float32),
    scratch_types=[
        pltpu.VMEM((_RCHUNK, D_MODEL), jnp.float32),
        pltpu.VMEM((1, D_MODEL), jnp.float32),
        pltpu.VMEM((_RCHUNK,), jnp.int32),
        pltpu.VMEM((_RCHUNK,), jnp.int32),
        pltpu.VMEM((_RCHUNK, D_MODEL), jnp.float32),
        pltpu.VMEM((_RCHUNK, D_MODEL), jnp.float32),
        pltpu.VMEM((_RCHUNK, D_MODEL), jnp.float32),
        pltpu.VMEM((_RCHUNK, D_MODEL), jnp.float32),
        pltpu.SemaphoreType.DMA,
        pltpu.SemaphoreType.DMA,
        pltpu.SemaphoreType.DMA,
        pltpu.SemaphoreType.DMA,
    ],
)(_sc_body)


def kernel(inputs, class_embed, pos_table):
    flat_in = inputs.reshape(BATCH * N_PATCHES, D_MODEL)
    cls = class_embed.reshape(1, D_MODEL)
    out = _sc_call(flat_in, cls, pos_table)
    return out
